# baseline (speedup 1.0000x reference)
import functools

import jax
import jax.numpy as jnp
from jax import lax
from jax.experimental import pallas as pl
from jax.experimental.pallas import tpu as pltpu
from jax.experimental.pallas import tpu_sc as plsc

NC = 2
NS = 16
NW = NC * NS
CHUNK = 128

_MESH = plsc.VectorSubcoreMesh(
    core_axis_name="c", subcore_axis_name="s", num_cores=NC, num_subcores=NS)
_SC_PARAMS = pltpu.CompilerParams(
    needs_layout_passes=False, use_tc_tiling_on_sc=False)


def _make_deg_kernel(n_pad, epw):
    @functools.partial(
        pl.kernel,
        out_type=jax.ShapeDtypeStruct((NW, n_pad), jnp.float32),
        mesh=_MESH,
        compiler_params=_SC_PARAMS,
        scratch_types=[
            pltpu.VMEM((epw,), jnp.int32),
            pltpu.VMEM((epw,), jnp.float32),
            pltpu.VMEM((n_pad,), jnp.float32),
        ],
    )
    def k(col_hbm, ew_hbm, out_hbm, col_v, ew_v, deg_v):
        cid = lax.axis_index("c")
        sid = lax.axis_index("s")
        wid = sid * NC + cid
        base = wid * epw
        pltpu.sync_copy(col_hbm.at[pl.ds(base, epw)], col_v)
        pltpu.sync_copy(ew_hbm.at[pl.ds(base, epw)], ew_v)
        zv = jnp.zeros((16,), jnp.float32)

        def zb(i, _):
            deg_v[pl.ds(i * 16, 16)] = zv
            return 0
        lax.fori_loop(0, n_pad // 16, zb, 0, unroll=4)

        def eb(i, _):
            c = col_v[pl.ds(i * 16, 16)]
            w = ew_v[pl.ds(i * 16, 16)]
            plsc.addupdate_scatter(deg_v, [c], w)
            return 0
        lax.fori_loop(0, epw // 16, eb, 0, unroll=4)
        pltpu.sync_copy(deg_v, out_hbm.at[wid])
    return k


def _make_agg_kernel(n_pad, kch, dout, tpb):
    dv = dout // 16

    @functools.partial(
        pl.kernel,
        out_type=jax.ShapeDtypeStruct((NC, n_pad, dout), jnp.float32),
        mesh=_MESH,
        compiler_params=_SC_PARAMS,
        scratch_types=[
            pltpu.VMEM((kch, CHUNK), jnp.int32),
            pltpu.VMEM((kch, CHUNK), jnp.int32),
            pltpu.VMEM((kch, CHUNK), jnp.float32),
            pltpu.VMEM((2, CHUNK, dout), jnp.float32),
            pltpu.VMEM((2, CHUNK, dout), jnp.float32),
            pltpu.VMEM_SHARED((n_pad, dout), jnp.float32),
            pltpu.SemaphoreType.DMA,
            pltpu.SemaphoreType.DMA,
            pltpu.SemaphoreType.DMA,
            pltpu.SemaphoreType.DMA,
            pltpu.SemaphoreType.DMA,
        ],
    )
    def k(p_hbm, row_hbm, col_hbm, ew_hbm, out_hbm,
          row_v, col_v, ew_v, gbuf, sbuf, acc,
          gsem0, gsem1, ssem0, ssem1, lsem):
        cid = lax.axis_index("c")
        sid = lax.axis_index("s")
        wid = sid * NC + cid
        gsems = (gsem0, gsem1)
        ssems = (ssem0, ssem1)
        ld_r = pltpu.async_copy(row_hbm.at[wid], row_v, lsem)
        ld_c = pltpu.async_copy(col_hbm.at[wid], col_v, lsem)
        ld_w = pltpu.async_copy(ew_hbm.at[wid], ew_v, lsem)

        zv = jnp.zeros((16,), jnp.float32)

        def zb(i, _):
            r = i // dv
            c = (i % dv) * 16
            sbuf[0, r, pl.ds(c, 16)] = zv
            return 0
        lax.fori_loop(0, CHUNK * dv, zb, 0, unroll=4)
        base = sid * tpb
        off = 0
        while off < tpb:
            sz = min(CHUNK, tpb - off)
            pltpu.sync_copy(sbuf.at[0, pl.ds(0, sz)],
                            acc.at[pl.ds(base + off, sz)])
            off += sz
        ld_r.wait()
        ld_c.wait()
        ld_w.wait()
        p_my = p_hbm.at[cid]
        pltpu.async_copy(p_my.at[row_v.at[0]], gbuf.at[0], gsem0)
        pltpu.async_copy(p_my.at[row_v.at[1]], gbuf.at[1], gsem1)
        plsc.subcore_barrier()

        def body(g, _):
            for b in range(2):
                j = g * 2 + b

                @pl.when(j < kch)
                def _():
                    pltpu.make_async_copy(
                        p_my.at[row_v.at[j]], gbuf.at[b], gsems[b]).wait()

                    @pl.when(j >= 2)
                    def _():
                        pltpu.make_async_copy(
                            sbuf.at[b], acc.at[col_v.at[j - 2]],
                            ssems[b]).wait()

                    def sc(gg, _):
                        wv = ew_v[j, pl.ds(gg * 16, 16)]
                        e0 = gg * 16
                        for l in range(16):
                            w = wv[l]
                            for c in range(dv):
                                sl = pl.ds(c * 16, 16)
                                sbuf[b, e0 + l, sl] = gbuf[b, e0 + l, sl] * w
                        return 0
                    lax.fori_loop(0, CHUNK // 16, sc, 0, unroll=4)

                    @pl.when(j + 2 < kch)
                    def _():
                        pltpu.async_copy(
                            p_my.at[row_v.at[j + 2]], gbuf.at[b], gsems[b])
                    pltpu.async_copy(
                        sbuf.at[b], acc.at[col_v.at[j]], ssems[b], add=True)
            return 0
        lax.fori_loop(0, (kch + 1) // 2, body, 0)
        b2, b1 = (kch - 2) % 2, (kch - 1) % 2
        pltpu.make_async_copy(
            sbuf.at[b2], acc.at[col_v.at[kch - 2]], ssems[b2]).wait()
        pltpu.make_async_copy(
            sbuf.at[b1], acc.at[col_v.at[kch - 1]], ssems[b1]).wait()
        plsc.subcore_barrier()

        off = 0
        while off < tpb:
            sz = min(CHUNK, tpb - off)
            pltpu.sync_copy(acc.at[pl.ds(base + off, sz)],
                            gbuf.at[0, pl.ds(0, sz)])
            pltpu.sync_copy(gbuf.at[0, pl.ds(0, sz)],
                            out_hbm.at[cid, pl.ds(base + off, sz)])
            off += sz
    return k


def _tc1_body(n, x_ref, w_ref, degt_ref, p_ref, dinv_ref):
    deg = jnp.sum(degt_ref[...], axis=1, keepdims=True) + 2.0
    dinv = lax.rsqrt(deg)[:n]
    dinv_ref[...] = dinv
    p = dinv * jnp.dot(x_ref[...], w_ref[...],
                       preferred_element_type=jnp.float32)
    p_ref[0] = p
    p_ref[1] = p


def _tcmid_body(n, pp_ref, pprev_ref, dinv_ref, b_ref, w_ref, pnext_ref):
    dinv = dinv_ref[...]
    agg = pp_ref[0, :n, :] + pp_ref[1, :n, :]
    h = dinv * agg + (2.0 * dinv) * pprev_ref[0] + b_ref[...]
    h = jnp.maximum(h, 0.0)
    p = dinv * jnp.dot(h, w_ref[...], preferred_element_type=jnp.float32)
    pnext_ref[0] = p
    pnext_ref[1] = p


def _tcfin_body(n, ncls, pp_ref, pprev_ref, dinv_ref, b_ref, out_ref):
    dinv = dinv_ref[...]
    agg = pp_ref[0, :n, :] + pp_ref[1, :n, :]
    h = dinv * agg + (2.0 * dinv) * pprev_ref[0] + b_ref[...]
    colid = lax.broadcasted_iota(jnp.int32, h.shape, 1)
    hm = jnp.where(colid < ncls, h, -1e30)
    m = jnp.max(hm, axis=1, keepdims=True)
    e = jnp.where(colid < ncls, jnp.exp(hm - m), 0.0)
    lse = jnp.log(jnp.sum(e, axis=1, keepdims=True))
    out_ref[...] = h - m - lse


def _tc_call(body, out_shape, *args):
    return pl.pallas_call(body, out_shape=out_shape)(*args)


def kernel(x, edge_index, edge_weight, W1, b1, W2, b2, W3, b3, W4, b4):
    n, d = x.shape
    e = edge_weight.shape[0]
    f32 = jnp.float32

    n_pad = -(-n // (NS * 8)) * (NS * 8)
    tpb = n_pad // NS
    kch = -(-e // (NW * CHUNK))
    e_pad = NW * CHUNK * kch
    epw = e_pad // NW

    row = jnp.pad(edge_index[0], (0, e_pad - e))
    col = jnp.pad(edge_index[1], (0, e_pad - e))
    ew = jnp.pad(edge_weight, (0, e_pad - e))
    row3 = row.reshape(NW, kch, CHUNK)
    col3 = col.reshape(NW, kch, CHUNK)
    ew3 = ew.reshape(NW, kch, CHUNK)

    dcls = W4.shape[1]
    dp = 16
    w4p = jnp.pad(W4, ((0, 0), (0, dp - dcls)))
    b4p = jnp.pad(b4, (0, dp - dcls))

    degp = _make_deg_kernel(n_pad, epw)(col, ew)
    degt = degp.T

    p1, dinv = _tc_call(
        functools.partial(_tc1_body, n),
        (jax.ShapeDtypeStruct((NC, n, W1.shape[1]), f32),
         jax.ShapeDtypeStruct((n, 1), f32)),
        x, W1, degt)

    agg64 = _make_agg_kernel(n_pad, kch, 64, tpb)
    agg32 = _make_agg_kernel(n_pad, kch, 32, tpb)
    agg16 = _make_agg_kernel(n_pad, kch, 16, tpb)

    pp1 = agg64(p1, row3, col3, ew3)
    p2 = _tc_call(functools.partial(_tcmid_body, n),
                  jax.ShapeDtypeStruct((NC, n, W2.shape[1]), f32),
                  pp1, p1, dinv, b1, W2)
    pp2 = agg32(p2, row3, col3, ew3)
    p3 = _tc_call(functools.partial(_tcmid_body, n),
                  jax.ShapeDtypeStruct((NC, n, W3.shape[1]), f32),
                  pp2, p2, dinv, b2, W3)
    pp3 = agg16(p3, row3, col3, ew3)
    p4 = _tc_call(functools.partial(_tcmid_body, n),
                  jax.ShapeDtypeStruct((NC, n, dp), f32),
                  pp3, p3, dinv, b3, w4p)
    pp4 = agg16(p4, row3, col3, ew3)
    out = _tc_call(functools.partial(_tcfin_body, n, dcls),
                   jax.ShapeDtypeStruct((n, dp), f32),
                   pp4, p4, dinv, b4p)
    return out[:, :dcls]


# speedup vs baseline: 1.1455x; 1.1455x over previous
import functools

import jax
import jax.numpy as jnp
from jax import lax
from jax.experimental import pallas as pl
from jax.experimental.pallas import tpu as pltpu
from jax.experimental.pallas import tpu_sc as plsc

NC = 2
NS = 16
NW = NC * NS
CHUNK = 128

_MESH = plsc.VectorSubcoreMesh(
    core_axis_name="c", subcore_axis_name="s", num_cores=NC, num_subcores=NS)
_SC_PARAMS = pltpu.CompilerParams(
    needs_layout_passes=False, use_tc_tiling_on_sc=False)


def _make_deg_kernel(n_pad, epw):
    @functools.partial(
        pl.kernel,
        out_type=jax.ShapeDtypeStruct((NW, n_pad), jnp.float32),
        mesh=_MESH,
        compiler_params=_SC_PARAMS,
        scratch_types=[
            pltpu.VMEM((epw,), jnp.int32),
            pltpu.VMEM((epw,), jnp.float32),
            pltpu.VMEM((n_pad,), jnp.float32),
        ],
    )
    def k(col_hbm, ew_hbm, out_hbm, col_v, ew_v, deg_v):
        cid = lax.axis_index("c")
        sid = lax.axis_index("s")
        wid = sid * NC + cid
        base = wid * epw
        pltpu.sync_copy(col_hbm.at[pl.ds(base, epw)], col_v)
        pltpu.sync_copy(ew_hbm.at[pl.ds(base, epw)], ew_v)
        zv = jnp.zeros((16,), jnp.float32)

        def zb(i, _):
            deg_v[pl.ds(i * 16, 16)] = zv
            return 0
        lax.fori_loop(0, n_pad // 16, zb, 0, unroll=4)

        def eb(i, _):
            c = col_v[pl.ds(i * 16, 16)]
            w = ew_v[pl.ds(i * 16, 16)]
            plsc.addupdate_scatter(deg_v, [c], w)
            return 0
        lax.fori_loop(0, epw // 16, eb, 0, unroll=4)
        pltpu.sync_copy(deg_v, out_hbm.at[wid])
    return k


def _make_agg_kernel(n_pad, k0, k1, dout, tpb):
    dv = dout // 16
    kmax = max(k0, k1)
    kpair = k0 + k1

    @functools.partial(
        pl.kernel,
        out_type=jax.ShapeDtypeStruct((NC, n_pad, dout), jnp.float32),
        mesh=_MESH,
        compiler_params=_SC_PARAMS,
        scratch_types=[
            pltpu.VMEM((kmax, CHUNK), jnp.int32),
            pltpu.VMEM((kmax, CHUNK), jnp.int32),
            pltpu.VMEM((kmax, CHUNK), jnp.float32),
            pltpu.VMEM((2, CHUNK, dout), jnp.float32),
            pltpu.VMEM((2, CHUNK, dout), jnp.float32),
            pltpu.VMEM_SHARED((n_pad, dout), jnp.float32),
            pltpu.SemaphoreType.DMA,
            pltpu.SemaphoreType.DMA,
            pltpu.SemaphoreType.DMA,
            pltpu.SemaphoreType.DMA,
            pltpu.SemaphoreType.DMA,
        ],
    )
    def k(p_hbm, row_hbm, col_hbm, ew_hbm, out_hbm,
          row_v, col_v, ew_v, gbuf, sbuf, acc,
          gsem0, gsem1, ssem0, ssem1, lsem):
        cid = lax.axis_index("c")
        sid = lax.axis_index("s")
        gsems = (gsem0, gsem1)
        ssems = (ssem0, ssem1)

        zv = jnp.zeros((16,), jnp.float32)

        def zb(i, _):
            r = i // dv
            c = (i % dv) * 16
            sbuf[0, r, pl.ds(c, 16)] = zv
            return 0
        lax.fori_loop(0, CHUNK * dv, zb, 0, unroll=4)
        base = sid * tpb
        off = 0
        while off < tpb:
            sz = min(CHUNK, tpb - off)
            pltpu.sync_copy(sbuf.at[0, pl.ds(0, sz)],
                            acc.at[pl.ds(base + off, sz)])
            off += sz

        def run_main(kch, coff):
            ld_r = pltpu.async_copy(
                row_hbm.at[sid, pl.ds(coff, kch)], row_v.at[pl.ds(0, kch)],
                lsem)
            ld_c = pltpu.async_copy(
                col_hbm.at[sid, pl.ds(coff, kch)], col_v.at[pl.ds(0, kch)],
                lsem)
            ld_w = pltpu.async_copy(
                ew_hbm.at[sid, pl.ds(coff, kch)], ew_v.at[pl.ds(0, kch)],
                lsem)
            ld_r.wait()
            ld_c.wait()
            ld_w.wait()
            pltpu.async_copy(p_hbm.at[row_v.at[0]], gbuf.at[0], gsem0)
            pltpu.async_copy(p_hbm.at[row_v.at[1]], gbuf.at[1], gsem1)
            plsc.subcore_barrier()

            def body(g, _):
                for b in range(2):
                    j = g * 2 + b

                    @pl.when(j < kch)
                    def _():
                        pltpu.make_async_copy(
                            p_hbm.at[row_v.at[j]], gbuf.at[b], gsems[b]).wait()

                        @pl.when(j >= 2)
                        def _():
                            pltpu.make_async_copy(
                                sbuf.at[b], acc.at[col_v.at[j - 2]],
                                ssems[b]).wait()

                        def sc(gg, _):
                            wv = ew_v[j, pl.ds(gg * 16, 16)]
                            e0 = gg * 16
                            for l in range(16):
                                w = wv[l]
                                for c in range(dv):
                                    sl = pl.ds(c * 16, 16)
                                    sbuf[b, e0 + l, sl] = (
                                        gbuf[b, e0 + l, sl] * w)
                            return 0
                        lax.fori_loop(0, CHUNK // 16, sc, 0, unroll=4)

                        @pl.when(j + 2 < kch)
                        def _():
                            pltpu.async_copy(
                                p_hbm.at[row_v.at[j + 2]], gbuf.at[b],
                                gsems[b])
                        pltpu.async_copy(
                            sbuf.at[b], acc.at[col_v.at[j]], ssems[b],
                            add=True)
                return 0
            lax.fori_loop(0, (kch + 1) // 2, body, 0)
            b2, b1 = (kch - 2) % 2, (kch - 1) % 2
            pltpu.make_async_copy(
                sbuf.at[b2], acc.at[col_v.at[kch - 2]], ssems[b2]).wait()
            pltpu.make_async_copy(
                sbuf.at[b1], acc.at[col_v.at[kch - 1]], ssems[b1]).wait()

        @pl.when(cid == 0)
        def _():
            run_main(k0, 0)

        @pl.when(cid == 1)
        def _():
            run_main(k1, k0)
        plsc.subcore_barrier()

        off = 0
        while off < tpb:
            sz = min(CHUNK, tpb - off)
            pltpu.sync_copy(acc.at[pl.ds(base + off, sz)],
                            gbuf.at[0, pl.ds(0, sz)])
            pltpu.sync_copy(gbuf.at[0, pl.ds(0, sz)],
                            out_hbm.at[cid, pl.ds(base + off, sz)])
            off += sz
    return k


def _tc1_body(n, x_ref, w_ref, degt_ref, p_ref, dinv_ref):
    deg = jnp.sum(degt_ref[...], axis=1, keepdims=True) + 2.0
    dinv = lax.rsqrt(deg)[:n]
    dinv_ref[...] = dinv
    p_ref[...] = dinv * jnp.dot(x_ref[...], w_ref[...],
                                preferred_element_type=jnp.float32)


def _tcmid_body(n, pp_ref, pprev_ref, dinv_ref, b_ref, w_ref, pnext_ref):
    dinv = dinv_ref[...]
    agg = pp_ref[0, :n, :] + pp_ref[1, :n, :]
    h = dinv * agg + (2.0 * dinv) * pprev_ref[...] + b_ref[...]
    h = jnp.maximum(h, 0.0)
    pnext_ref[...] = dinv * jnp.dot(h, w_ref[...],
                                    preferred_element_type=jnp.float32)


def _tcfin_body(n, ncls, pp_ref, pprev_ref, dinv_ref, b_ref, out_ref):
    dinv = dinv_ref[...]
    agg = pp_ref[0, :n, :] + pp_ref[1, :n, :]
    h = dinv * agg + (2.0 * dinv) * pprev_ref[...] + b_ref[...]
    colid = lax.broadcasted_iota(jnp.int32, h.shape, 1)
    hm = jnp.where(colid < ncls, h, -1e30)
    m = jnp.max(hm, axis=1, keepdims=True)
    e = jnp.where(colid < ncls, jnp.exp(hm - m), 0.0)
    lse = jnp.log(jnp.sum(e, axis=1, keepdims=True))
    out_ref[...] = h - m - lse


def _tc_call(body, out_shape, *args):
    return pl.pallas_call(body, out_shape=out_shape)(*args)


def kernel(x, edge_index, edge_weight, W1, b1, W2, b2, W3, b3, W4, b4):
    n, d = x.shape
    e = edge_weight.shape[0]
    f32 = jnp.float32

    n_pad = -(-n // (NS * 8)) * (NS * 8)
    tpb = n_pad // NS
    kpair = -(-e // (NS * CHUNK))
    k0 = (kpair * 3) // 4
    k1 = kpair - k0
    e_pad = NS * CHUNK * kpair
    epw = e_pad // NW

    row = jnp.pad(edge_index[0], (0, e_pad - e))
    col = jnp.pad(edge_index[1], (0, e_pad - e))
    ew = jnp.pad(edge_weight, (0, e_pad - e))
    row3 = row.reshape(NS, kpair, CHUNK)
    col3 = col.reshape(NS, kpair, CHUNK)
    ew3 = ew.reshape(NS, kpair, CHUNK)

    dcls = W4.shape[1]
    dp = 16
    w4p = jnp.pad(W4, ((0, 0), (0, dp - dcls)))
    b4p = jnp.pad(b4, (0, dp - dcls))

    degp = _make_deg_kernel(n_pad, epw)(col, ew)
    degt = degp.T

    p1, dinv = _tc_call(
        functools.partial(_tc1_body, n),
        (jax.ShapeDtypeStruct((n, W1.shape[1]), f32),
         jax.ShapeDtypeStruct((n, 1), f32)),
        x, W1, degt)

    agg64 = _make_agg_kernel(n_pad, k0, k1, 64, tpb)
    agg32 = _make_agg_kernel(n_pad, k0, k1, 32, tpb)
    agg16 = _make_agg_kernel(n_pad, k0, k1, 16, tpb)

    pp1 = agg64(p1, row3, col3, ew3)
    p2 = _tc_call(functools.partial(_tcmid_body, n),
                  jax.ShapeDtypeStruct((n, W2.shape[1]), f32),
                  pp1, p1, dinv, b1, W2)
    pp2 = agg32(p2, row3, col3, ew3)
    p3 = _tc_call(functools.partial(_tcmid_body, n),
                  jax.ShapeDtypeStruct((n, W3.shape[1]), f32),
                  pp2, p2, dinv, b2, W3)
    pp3 = agg16(p3, row3, col3, ew3)
    p4 = _tc_call(functools.partial(_tcmid_body, n),
                  jax.ShapeDtypeStruct((n, dp), f32),
                  pp3, p3, dinv, b3, w4p)
    pp4 = agg16(p4, row3, col3, ew3)
    out = _tc_call(functools.partial(_tcfin_body, n, dcls),
                   jax.ShapeDtypeStruct((n, dp), f32),
                   pp4, p4, dinv, b4p)
    return out[:, :dcls]


# speedup vs baseline: 1.2552x; 1.0958x over previous
import functools

import jax
import jax.numpy as jnp
from jax import lax
from jax.experimental import pallas as pl
from jax.experimental.pallas import tpu as pltpu
from jax.experimental.pallas import tpu_sc as plsc

NC = 2
NS = 16
NW = NC * NS
CHUNK = 128

_MESH = plsc.VectorSubcoreMesh(
    core_axis_name="c", subcore_axis_name="s", num_cores=NC, num_subcores=NS)
_SC_PARAMS = pltpu.CompilerParams(
    needs_layout_passes=False, use_tc_tiling_on_sc=False)


def _make_deg_kernel(n_pad, epw):
    @functools.partial(
        pl.kernel,
        out_type=jax.ShapeDtypeStruct((NW, n_pad), jnp.float32),
        mesh=_MESH,
        compiler_params=_SC_PARAMS,
        scratch_types=[
            pltpu.VMEM((epw,), jnp.int32),
            pltpu.VMEM((epw,), jnp.float32),
            pltpu.VMEM((n_pad,), jnp.float32),
        ],
    )
    def k(col_hbm, ew_hbm, out_hbm, col_v, ew_v, deg_v):
        cid = lax.axis_index("c")
        sid = lax.axis_index("s")
        wid = sid * NC + cid
        base = wid * epw
        pltpu.sync_copy(col_hbm.at[pl.ds(base, epw)], col_v)
        pltpu.sync_copy(ew_hbm.at[pl.ds(base, epw)], ew_v)
        zv = jnp.zeros((16,), jnp.float32)

        def zb(i, _):
            deg_v[pl.ds(i * 16, 16)] = zv
            return 0
        lax.fori_loop(0, n_pad // 16, zb, 0, unroll=4)

        def eb(i, _):
            c = col_v[pl.ds(i * 16, 16)]
            w = ew_v[pl.ds(i * 16, 16)]
            plsc.addupdate_scatter(deg_v, [c], w)
            return 0
        lax.fori_loop(0, epw // 16, eb, 0, unroll=4)
        pltpu.sync_copy(deg_v, out_hbm.at[wid])
    return k


def _make_agg_kernel(n_pad, k0, k1, dout, tpb):
    dv = dout // 16
    kmax = max(k0, k1)
    kpair = k0 + k1

    @functools.partial(
        pl.kernel,
        out_type=jax.ShapeDtypeStruct((NC, n_pad, dout), jnp.float32),
        mesh=_MESH,
        compiler_params=_SC_PARAMS,
        scratch_types=[
            pltpu.VMEM((kmax, CHUNK), jnp.int32),
            pltpu.VMEM((kmax, CHUNK), jnp.int32),
            pltpu.VMEM((kmax, CHUNK), jnp.float32),
            pltpu.VMEM((2, CHUNK, dout), jnp.float32),
            pltpu.VMEM((2, CHUNK, dout), jnp.float32),
            pltpu.VMEM_SHARED((n_pad, dout), jnp.float32),
            pltpu.SemaphoreType.DMA,
            pltpu.SemaphoreType.DMA,
            pltpu.SemaphoreType.DMA,
            pltpu.SemaphoreType.DMA,
            pltpu.SemaphoreType.DMA,
        ],
    )
    def k(p_hbm, row_hbm, col_hbm, ew_hbm, out_hbm,
          row_v, col_v, ew_v, gbuf, sbuf, acc,
          gsem0, gsem1, ssem0, ssem1, lsem):
        cid = lax.axis_index("c")
        sid = lax.axis_index("s")
        gsems = (gsem0, gsem1)
        ssems = (ssem0, ssem1)

        zv = jnp.zeros((16,), jnp.float32)

        def zb(i, _):
            r = i // dv
            c = (i % dv) * 16
            sbuf[0, r, pl.ds(c, 16)] = zv
            return 0
        lax.fori_loop(0, CHUNK * dv, zb, 0, unroll=4)
        base = sid * tpb
        off = 0
        while off < tpb:
            sz = min(CHUNK, tpb - off)
            pltpu.sync_copy(sbuf.at[0, pl.ds(0, sz)],
                            acc.at[pl.ds(base + off, sz)])
            off += sz

        def run_main(kch, coff):
            ld_r = pltpu.async_copy(
                row_hbm.at[sid, pl.ds(coff, kch)], row_v.at[pl.ds(0, kch)],
                lsem)
            ld_c = pltpu.async_copy(
                col_hbm.at[sid, pl.ds(coff, kch)], col_v.at[pl.ds(0, kch)],
                lsem)
            ld_w = pltpu.async_copy(
                ew_hbm.at[sid, pl.ds(coff, kch)], ew_v.at[pl.ds(0, kch)],
                lsem)
            ld_r.wait()
            ld_c.wait()
            ld_w.wait()
            pltpu.async_copy(p_hbm.at[row_v.at[0]], gbuf.at[0], gsem0)
            pltpu.async_copy(p_hbm.at[row_v.at[1]], gbuf.at[1], gsem1)
            plsc.subcore_barrier()

            def body(g, _):
                for b in range(2):
                    j = g * 2 + b

                    @pl.when(j < kch)
                    def _():
                        pltpu.make_async_copy(
                            p_hbm.at[row_v.at[j]], gbuf.at[b], gsems[b]).wait()

                        @pl.when(j >= 2)
                        def _():
                            pltpu.make_async_copy(
                                sbuf.at[b], acc.at[col_v.at[j - 2]],
                                ssems[b]).wait()

                        def sc(gg, _):
                            wv = ew_v[j, pl.ds(gg * 16, 16)]
                            e0 = gg * 16
                            for l in range(16):
                                w = wv[l]
                                for c in range(dv):
                                    sl = pl.ds(c * 16, 16)
                                    sbuf[b, e0 + l, sl] = (
                                        gbuf[b, e0 + l, sl] * w)
                            return 0
                        lax.fori_loop(0, CHUNK // 16, sc, 0, unroll=4)

                        @pl.when(j + 2 < kch)
                        def _():
                            pltpu.async_copy(
                                p_hbm.at[row_v.at[j + 2]], gbuf.at[b],
                                gsems[b])
                        pltpu.async_copy(
                            sbuf.at[b], acc.at[col_v.at[j]], ssems[b],
                            add=True)
                return 0
            lax.fori_loop(0, (kch + 1) // 2, body, 0)
            b2, b1 = (kch - 2) % 2, (kch - 1) % 2
            pltpu.make_async_copy(
                sbuf.at[b2], acc.at[col_v.at[kch - 2]], ssems[b2]).wait()
            pltpu.make_async_copy(
                sbuf.at[b1], acc.at[col_v.at[kch - 1]], ssems[b1]).wait()

        @pl.when(cid == 0)
        def _():
            run_main(k0, 0)

        @pl.when(cid == 1)
        def _():
            run_main(k1, k0)
        plsc.subcore_barrier()

        off = 0
        while off < tpb:
            sz = min(CHUNK, tpb - off)
            pltpu.sync_copy(acc.at[pl.ds(base + off, sz)],
                            gbuf.at[0, pl.ds(0, sz)])
            pltpu.sync_copy(gbuf.at[0, pl.ds(0, sz)],
                            out_hbm.at[cid, pl.ds(base + off, sz)])
            off += sz
    return k


def _tc1_body(n, x_ref, w_ref, degt_ref, p_ref, dinv_ref):
    deg = jnp.sum(degt_ref[...], axis=1, keepdims=True) + 2.0
    dinv = lax.rsqrt(deg)[:n]
    dinv_ref[...] = dinv
    p_ref[...] = dinv * jnp.dot(x_ref[...], w_ref[...],
                                preferred_element_type=jnp.float32)


def _tcmid_body(n, pp_ref, pprev_ref, dinv_ref, b_ref, w_ref, pnext_ref):
    dinv = dinv_ref[...]
    agg = pp_ref[0, :n, :] + pp_ref[1, :n, :]
    h = dinv * agg + (2.0 * dinv) * pprev_ref[...] + b_ref[...]
    h = jnp.maximum(h, 0.0)
    pnext_ref[...] = dinv * jnp.dot(h, w_ref[...],
                                    preferred_element_type=jnp.float32)


def _tcfin_body(n, ncls, pp_ref, pprev_ref, dinv_ref, b_ref, out_ref):
    dinv = dinv_ref[...]
    agg = pp_ref[0, :n, :] + pp_ref[1, :n, :]
    h = dinv * agg + (2.0 * dinv) * pprev_ref[...] + b_ref[...]
    colid = lax.broadcasted_iota(jnp.int32, h.shape, 1)
    hm = jnp.where(colid < ncls, h, -1e30)
    m = jnp.max(hm, axis=1, keepdims=True)
    e = jnp.where(colid < ncls, jnp.exp(hm - m), 0.0)
    lse = jnp.log(jnp.sum(e, axis=1, keepdims=True))
    out_ref[...] = h - m - lse


def _tc_call(body, out_shape, *args):
    return pl.pallas_call(body, out_shape=out_shape)(*args)


def kernel(x, edge_index, edge_weight, W1, b1, W2, b2, W3, b3, W4, b4):
    n, d = x.shape
    e = edge_weight.shape[0]
    f32 = jnp.float32

    n_pad = -(-n // (NS * 8)) * (NS * 8)
    tpb = n_pad // NS
    kpair = -(-e // (NS * CHUNK))
    e_pad = NS * CHUNK * kpair
    epw = e_pad // NW

    row = jnp.pad(edge_index[0], (0, e_pad - e))
    col = jnp.pad(edge_index[1], (0, e_pad - e))
    ew = jnp.pad(edge_weight, (0, e_pad - e))
    row3 = row.reshape(NS, kpair, CHUNK)
    col3 = col.reshape(NS, kpair, CHUNK)
    ew3 = ew.reshape(NS, kpair, CHUNK)

    dcls = W4.shape[1]
    dp = 16
    w4p = jnp.pad(W4, ((0, 0), (0, dp - dcls)))
    b4p = jnp.pad(b4, (0, dp - dcls))

    degp = _make_deg_kernel(n_pad, epw)(col, ew)
    degt = degp.T

    p1, dinv = _tc_call(
        functools.partial(_tc1_body, n),
        (jax.ShapeDtypeStruct((n, W1.shape[1]), f32),
         jax.ShapeDtypeStruct((n, 1), f32)),
        x, W1, degt)

    k64 = (kpair * 111 + 156) // 157
    k32 = (kpair * 96 + 156) // 157
    k16 = (kpair * 90 + 156) // 157
    agg64 = _make_agg_kernel(n_pad, k64, kpair - k64, 64, tpb)
    agg32 = _make_agg_kernel(n_pad, k32, kpair - k32, 32, tpb)
    agg16 = _make_agg_kernel(n_pad, k16, kpair - k16, 16, tpb)

    pp1 = agg64(p1, row3, col3, ew3)
    p2 = _tc_call(functools.partial(_tcmid_body, n),
                  jax.ShapeDtypeStruct((n, W2.shape[1]), f32),
                  pp1, p1, dinv, b1, W2)
    pp2 = agg32(p2, row3, col3, ew3)
    p3 = _tc_call(functools.partial(_tcmid_body, n),
                  jax.ShapeDtypeStruct((n, W3.shape[1]), f32),
                  pp2, p2, dinv, b2, W3)
    pp3 = agg16(p3, row3, col3, ew3)
    p4 = _tc_call(functools.partial(_tcmid_body, n),
                  jax.ShapeDtypeStruct((n, dp), f32),
                  pp3, p3, dinv, b3, w4p)
    pp4 = agg16(p4, row3, col3, ew3)
    out = _tc_call(functools.partial(_tcfin_body, n, dcls),
                   jax.ShapeDtypeStruct((n, dp), f32),
                   pp4, p4, dinv, b4p)
    return out[:, :dcls]
